# baseline (speedup 1.0000x reference)
import functools

import jax
import jax.numpy as jnp
from jax import lax
from jax.experimental import pallas as pl
from jax.experimental.pallas import tpu as pltpu
from jax.experimental.pallas import tpu_sc as plsc

_NUM_WORKERS = 32


def _tc_body(x_ref, wt_ref, b_ref, g_ref, ones_ref,
             z0_ref, z1_ref, z2_ref, z3_ref, s_ref, m_ref, u_ref, p_ref):
    x = x_ref[...]
    dot5 = jnp.dot(x, wt_ref[...], preferred_element_type=jnp.float32)
    logits = dot5[:, 0:4]
    s = dot5[:, 4:5]
    z = (logits + b_ref[...]) + g_ref[...]
    z0_ref[...] = z[:, 0:1]
    z1_ref[...] = z[:, 1:2]
    z2_ref[...] = z[:, 2:3]
    z3_ref[...] = z[:, 3:4]

    xb = x.astype(jnp.bfloat16)
    m_ref[...] = jnp.max(xb, axis=1, keepdims=True).astype(jnp.float32)

    ones_col = ones_ref[...]
    one_b = jnp.bfloat16(1.0)
    zero_b = jnp.bfloat16(0.0)

    ind_u = jnp.where(xb != zero_b, one_b, zero_b)
    u_ref[...] = jnp.dot(ind_u, ones_col, preferred_element_type=jnp.float32)

    xr = pltpu.roll(xb, 1, 1)
    ind_p = jnp.where(xb != xr, one_b, zero_b)
    pat_raw = jnp.dot(ind_p, ones_col, preferred_element_type=jnp.float32)
    wrap = jnp.where(xb[:, 0:1].astype(jnp.float32) != xb[:, -1:].astype(jnp.float32),
                     1.0, 0.0)
    p_ref[...] = pat_raw - wrap
    s_ref[...] = s


def _make_sc_combine(n):
    rpw = n // _NUM_WORKERS
    steps = rpw // 16
    mesh = plsc.VectorSubcoreMesh(core_axis_name="c", subcore_axis_name="s")

    @functools.partial(
        pl.kernel,
        out_type=jax.ShapeDtypeStruct((n,), jnp.float32),
        mesh=mesh,
        scratch_types=[
            pltpu.VMEM((rpw,), jnp.float32),
            pltpu.VMEM((rpw,), jnp.float32),
            pltpu.VMEM((rpw,), jnp.float32),
            pltpu.VMEM((rpw,), jnp.float32),
            pltpu.VMEM((rpw,), jnp.float32),
            pltpu.VMEM((rpw,), jnp.float32),
            pltpu.VMEM((rpw,), jnp.float32),
            pltpu.VMEM((rpw,), jnp.float32),
            pltpu.VMEM((rpw,), jnp.float32),
        ],
    )
    def combine(z0_hbm, z1_hbm, z2_hbm, z3_hbm, s_hbm, m_hbm, u_hbm, p_hbm,
                out_hbm,
                z0_v, z1_v, z2_v, z3_v, s_v, m_v, u_v, p_v, o_v):
        wid = lax.axis_index("s") * 2 + lax.axis_index("c")
        base = wid * rpw
        pltpu.sync_copy(z0_hbm.at[pl.ds(base, rpw)], z0_v)
        pltpu.sync_copy(z1_hbm.at[pl.ds(base, rpw)], z1_v)
        pltpu.sync_copy(z2_hbm.at[pl.ds(base, rpw)], z2_v)
        pltpu.sync_copy(z3_hbm.at[pl.ds(base, rpw)], z3_v)
        pltpu.sync_copy(s_hbm.at[pl.ds(base, rpw)], s_v)
        pltpu.sync_copy(m_hbm.at[pl.ds(base, rpw)], m_v)
        pltpu.sync_copy(u_hbm.at[pl.ds(base, rpw)], u_v)
        pltpu.sync_copy(p_hbm.at[pl.ds(base, rpw)], p_v)
        for i in range(steps):
            sl16 = pl.ds(i * 16, 16)
            z0 = z0_v[sl16]
            z1 = z1_v[sl16]
            z2 = z2_v[sl16]
            z3 = z3_v[sl16]
            sl = s_v[sl16]
            fl = m_v[sl16] / (sl + 1e-6)
            ul = u_v[sl16]
            pt = p_v[sl16]
            best, out = z0, sl
            for ze, ve in ((z1, fl), (z2, ul), (z3, pt)):
                take = ze > best
                best = jnp.where(take, ze, best)
                out = jnp.where(take, ve, out)
            o_v[sl16] = out
        pltpu.sync_copy(o_v, out_hbm.at[pl.ds(base, rpw)])

    return combine


def kernel(histograms, W, b, gumbel):
    n, d = histograms.shape
    e = W.shape[0]
    bm = min(1024, n)
    wt_aug = jnp.concatenate([W.T, jnp.ones((d, 1), jnp.float32)], axis=1)
    ones_col = jnp.ones((d, 1), jnp.bfloat16)
    col = jax.ShapeDtypeStruct((n, 1), jnp.float32)
    cols = pl.pallas_call(
        _tc_body,
        grid=(n // bm,),
        in_specs=[
            pl.BlockSpec((bm, d), lambda i: (i, 0)),
            pl.BlockSpec((d, e + 1), lambda i: (0, 0)),
            pl.BlockSpec((1, e), lambda i: (0, 0)),
            pl.BlockSpec((bm, e), lambda i: (i, 0)),
            pl.BlockSpec((d, 1), lambda i: (0, 0)),
        ],
        out_specs=[pl.BlockSpec((bm, 1), lambda i: (i, 0))] * 8,
        out_shape=[col] * 8,
    )(histograms, wt_aug, b.reshape(1, e), gumbel, ones_col)
    combine = _make_sc_combine(n)
    return combine(*(c.reshape(n) for c in cols))


# speedup vs baseline: 1.1527x; 1.1527x over previous
import jax
import jax.numpy as jnp
from jax.experimental import pallas as pl
from jax.experimental.pallas import tpu as pltpu


def _stats_body(x_ref, wt_ref, b_ref, g_ref, ones_ref,
                z0_ref, z1_ref, z2_ref, z3_ref, s_ref, m_ref, u_ref, p_ref):
    x = x_ref[...]
    dot5 = jnp.dot(x, wt_ref[...], preferred_element_type=jnp.float32)
    logits = dot5[:, 0:4]
    s = dot5[:, 4:5]
    z = (logits + b_ref[...]) + g_ref[...]
    z0_ref[...] = z[:, 0:1]
    z1_ref[...] = z[:, 1:2]
    z2_ref[...] = z[:, 2:3]
    z3_ref[...] = z[:, 3:4]

    xb = x.astype(jnp.bfloat16)
    m_ref[...] = jnp.max(xb, axis=1, keepdims=True).astype(jnp.float32)

    ones_col = ones_ref[...]
    one_b = jnp.bfloat16(1.0)
    zero_b = jnp.bfloat16(0.0)

    ind_u = jnp.where(xb != zero_b, one_b, zero_b)
    u_ref[...] = jnp.dot(ind_u, ones_col, preferred_element_type=jnp.float32)

    xr = pltpu.roll(xb, 1, 1)
    ind_p = jnp.where(xb != xr, one_b, zero_b)
    pat_raw = jnp.dot(ind_p, ones_col, preferred_element_type=jnp.float32)
    wrap = jnp.where(xb[:, 0:1].astype(jnp.float32) != xb[:, -1:].astype(jnp.float32),
                     1.0, 0.0)
    p_ref[...] = pat_raw - wrap
    s_ref[...] = s


def _combine_body(z0_ref, z1_ref, z2_ref, z3_ref, s_ref, m_ref, u_ref, p_ref,
                  out_ref):
    s = s_ref[...]
    freq = m_ref[...] / (s + 1e-6)
    best = z0_ref[...]
    out = s
    for z_ref, val in ((z1_ref, freq), (z2_ref, u_ref[...]),
                       (z3_ref, p_ref[...])):
        ze = z_ref[...]
        take = ze > best
        best = jnp.where(take, ze, best)
        out = jnp.where(take, val, out)
    out_ref[...] = out


def kernel(histograms, W, b, gumbel):
    n, d = histograms.shape
    e = W.shape[0]
    bm = min(1024, n)
    wt_aug = jnp.concatenate([W.T, jnp.ones((d, 1), jnp.float32)], axis=1)
    ones_col = jnp.ones((d, 1), jnp.bfloat16)
    col = jax.ShapeDtypeStruct((n, 1), jnp.float32)
    cols = pl.pallas_call(
        _stats_body,
        grid=(n // bm,),
        in_specs=[
            pl.BlockSpec((bm, d), lambda i: (i, 0)),
            pl.BlockSpec((d, e + 1), lambda i: (0, 0)),
            pl.BlockSpec((1, e), lambda i: (0, 0)),
            pl.BlockSpec((bm, e), lambda i: (i, 0)),
            pl.BlockSpec((d, 1), lambda i: (0, 0)),
        ],
        out_specs=[pl.BlockSpec((bm, 1), lambda i: (i, 0))] * 8,
        out_shape=[col] * 8,
    )(histograms, wt_aug, b.reshape(1, e), gumbel, ones_col)

    rows = max(8, n // 128)
    tiles = [c.reshape(rows, n // rows) for c in cols]
    out = pl.pallas_call(
        _combine_body,
        out_shape=jax.ShapeDtypeStruct((rows, n // rows), jnp.float32),
    )(*tiles)
    return out.reshape(n)


# speedup vs baseline: 1.6800x; 1.4575x over previous
import jax
import jax.numpy as jnp
from jax.experimental import pallas as pl
from jax.experimental.pallas import tpu as pltpu


def _body(x_ref, wt_ref, b_ref, g_ref, ones_ref, out_ref):
    x = x_ref[...]
    dot5 = jnp.dot(x, wt_ref[...], preferred_element_type=jnp.float32)
    logits = dot5[:, 0:4]
    s = dot5[:, 4:5]
    z = (logits + b_ref[...]) + g_ref[...]

    xb = x.astype(jnp.bfloat16)
    m = jnp.max(xb, axis=1, keepdims=True).astype(jnp.float32)

    ones_col = ones_ref[...]
    one_b = jnp.bfloat16(1.0)
    zero_b = jnp.bfloat16(0.0)

    ind_u = jnp.where(xb != zero_b, one_b, zero_b)
    uniq = jnp.dot(ind_u, ones_col, preferred_element_type=jnp.float32)

    xr = pltpu.roll(xb, 1, 1)
    ind_p = jnp.where(xb != xr, one_b, zero_b)
    pat_raw = jnp.dot(ind_p, ones_col, preferred_element_type=jnp.float32)
    wrap = jnp.where(xb[:, 0:1].astype(jnp.float32) != xb[:, -1:].astype(jnp.float32),
                     1.0, 0.0)
    pat = pat_raw - wrap

    freq = m / (s + 1e-6)

    vals = (s, freq, uniq, pat)
    best = z[:, 0:1]
    out = vals[0]
    for e in range(1, 4):
        ze = z[:, e:e + 1]
        take = ze > best
        best = jnp.where(take, ze, best)
        out = jnp.where(take, vals[e], out)
    out_ref[...] = out


def kernel(histograms, W, b, gumbel):
    n, d = histograms.shape
    e = W.shape[0]
    bm = min(1024, n)
    wt_aug = jnp.concatenate([W.T, jnp.ones((d, 1), jnp.float32)], axis=1)
    ones_col = jnp.ones((d, 1), jnp.bfloat16)
    out = pl.pallas_call(
        _body,
        grid=(n // bm,),
        in_specs=[
            pl.BlockSpec((bm, d), lambda i: (i, 0)),
            pl.BlockSpec((d, e + 1), lambda i: (0, 0)),
            pl.BlockSpec((1, e), lambda i: (0, 0)),
            pl.BlockSpec((bm, e), lambda i: (i, 0)),
            pl.BlockSpec((d, 1), lambda i: (0, 0)),
        ],
        out_specs=pl.BlockSpec((bm, 1), lambda i: (i, 0)),
        out_shape=jax.ShapeDtypeStruct((n, 1), jnp.float32),
    )(histograms, wt_aug, b.reshape(1, e), gumbel, ones_col)
    return out[:, 0]
